# baseline (speedup 1.0000x reference)
import functools

import jax
import jax.numpy as jnp
from jax import lax
from jax.experimental import pallas as pl
from jax.experimental.pallas import tpu as pltpu
from jax.experimental.pallas import tpu_sc as plsc

B, L, D = 4, 2048, 1024
SC_BATCHES = 1
TC_B = B - SC_BATCHES
NC, NS = 2, 16
NW = NC * NS
SC_ROWS = SC_BATCHES * L
ROWS_W = SC_ROWS // NW
CH = 16
CHW = CH * D
SC_BASE = TC_B * L


def _tc_body(x_ref, e_ref, o_ref):
    o_ref[...] = x_ref[...] + e_ref[...]


def _tc_add(inputs, emb3):
    BL = 512
    return pl.pallas_call(
        _tc_body,
        grid=(L // BL,),
        in_specs=[
            pl.BlockSpec((TC_B, BL, D), lambda j: (0, j, 0)),
            pl.BlockSpec((1, BL, D), lambda j: (0, j, 0)),
        ],
        out_specs=pl.BlockSpec((TC_B, BL, D), lambda j: (0, j, 0)),
        out_shape=jax.ShapeDtypeStruct((TC_B, L, D), inputs.dtype),
    )(inputs, emb3)


@functools.partial(
    pl.kernel,
    out_type=jax.ShapeDtypeStruct((SC_ROWS * D,), jnp.float32),
    mesh=plsc.VectorSubcoreMesh(core_axis_name="c", subcore_axis_name="s"),
    scratch_types=[
        pltpu.VMEM((CHW,), jnp.float32),
        pltpu.VMEM((CHW,), jnp.float32),
    ],
)
def _sc_add(in_hbm, emb_hbm, out_hbm, xbuf, ebuf):
    wid = lax.axis_index("s") * NC + lax.axis_index("c")
    rbase = wid * ROWS_W

    @pl.loop(0, ROWS_W // CH)
    def _chunk(ci):
        row = rbase + ci * CH
        pltpu.sync_copy(in_hbm.at[pl.ds((SC_BASE + row) * D, CHW)], xbuf)
        pltpu.sync_copy(emb_hbm.at[pl.ds(row * D, CHW)], ebuf)

        @pl.loop(0, CHW // 16, unroll=8)
        def _add(i):
            s = pl.ds(i * 16, 16)
            xbuf[s] = xbuf[s] + ebuf[s]

        pltpu.sync_copy(xbuf, out_hbm.at[pl.ds(row * D, CHW)])


def _tc_full(inputs, emb3, BL=256):
    return pl.pallas_call(
        _tc_body,
        grid=(L // BL,),
        in_specs=[
            pl.BlockSpec((B, BL, D), lambda j: (0, j, 0)),
            pl.BlockSpec((1, BL, D), lambda j: (0, j, 0)),
        ],
        out_specs=pl.BlockSpec((B, BL, D), lambda j: (0, j, 0)),
        out_shape=jax.ShapeDtypeStruct((B, L, D), inputs.dtype),
    )(inputs, emb3)


POS_W = L // NW
SCCH = 8
NCHUNK = POS_W // SCCH
NBUF = 3


@functools.partial(
    pl.kernel,
    out_type=jax.ShapeDtypeStruct((B, L, D), jnp.float32),
    mesh=plsc.VectorSubcoreMesh(core_axis_name="c", subcore_axis_name="s"),
    scratch_types=[
        pltpu.VMEM((NBUF, B, SCCH, D), jnp.float32),
        pltpu.VMEM((NBUF, SCCH, D), jnp.float32),
        pltpu.SemaphoreType.DMA((NBUF,)),
        pltpu.SemaphoreType.DMA((NBUF,)),
        pltpu.SemaphoreType.DMA((NBUF,)),
    ],
)
def _sc_full(in_hbm, emb_hbm, out_hbm, xbuf, ebuf, xsem, esem, osem):
    wid = lax.axis_index("s") * NC + lax.axis_index("c")
    wbase = wid * POS_W

    xd = [None] * NBUF
    ed = [None] * NBUF
    od = [None] * NBUF

    def start_in(g):
        slot = g % NBUF
        pos = wbase + g * SCCH
        ed[slot] = pltpu.async_copy(
            emb_hbm.at[pl.ds(pos, SCCH)], ebuf.at[slot], esem.at[slot])
        xd[slot] = pltpu.async_copy(
            in_hbm.at[:, pl.ds(pos, SCCH), :], xbuf.at[slot], xsem.at[slot])

    for g in range(min(NBUF - 1, NCHUNK)):
        start_in(g)

    for g in range(NCHUNK):
        slot = g % NBUF
        nxt = g + NBUF - 1
        if nxt < NCHUNK:
            nslot = nxt % NBUF
            if od[nslot] is not None:
                od[nslot].wait()
            start_in(nxt)
        ed[slot].wait()
        xd[slot].wait()

        @pl.loop(0, SCCH)
        def _row(j):
            @pl.loop(0, D // 16, unroll=8)
            def _vec(k):
                s = pl.ds(k * 16, 16)
                e = ebuf[slot, j, s]
                for b in range(B):
                    xbuf[slot, b, j, s] = xbuf[slot, b, j, s] + e

        pos = wbase + g * SCCH
        od[slot] = pltpu.async_copy(
            xbuf.at[slot], out_hbm.at[:, pl.ds(pos, SCCH), :], osem.at[slot])

    for g in range(max(0, NCHUNK - NBUF), NCHUNK):
        od[g % NBUF].wait()


def kernel(inputs, emb_table):
    return _sc_full(inputs, emb_table)


# speedup vs baseline: 1.0662x; 1.0662x over previous
import functools

import jax
import jax.numpy as jnp
from jax import lax
from jax.experimental import pallas as pl
from jax.experimental.pallas import tpu as pltpu
from jax.experimental.pallas import tpu_sc as plsc

B, L, D = 4, 2048, 1024
SC_BATCHES = 1
TC_B = B - SC_BATCHES
NC, NS = 2, 16
NW = NC * NS
SC_ROWS = SC_BATCHES * L
ROWS_W = SC_ROWS // NW
CH = 16
CHW = CH * D
SC_BASE = TC_B * L


def _tc_body(x_ref, e_ref, o_ref):
    o_ref[...] = x_ref[...] + e_ref[...]


def _tc_add(inputs, emb3):
    BL = 512
    return pl.pallas_call(
        _tc_body,
        grid=(L // BL,),
        in_specs=[
            pl.BlockSpec((TC_B, BL, D), lambda j: (0, j, 0)),
            pl.BlockSpec((1, BL, D), lambda j: (0, j, 0)),
        ],
        out_specs=pl.BlockSpec((TC_B, BL, D), lambda j: (0, j, 0)),
        out_shape=jax.ShapeDtypeStruct((TC_B, L, D), inputs.dtype),
    )(inputs, emb3)


@functools.partial(
    pl.kernel,
    out_type=jax.ShapeDtypeStruct((SC_ROWS * D,), jnp.float32),
    mesh=plsc.VectorSubcoreMesh(core_axis_name="c", subcore_axis_name="s"),
    scratch_types=[
        pltpu.VMEM((CHW,), jnp.float32),
        pltpu.VMEM((CHW,), jnp.float32),
    ],
)
def _sc_add(in_hbm, emb_hbm, out_hbm, xbuf, ebuf):
    wid = lax.axis_index("s") * NC + lax.axis_index("c")
    rbase = wid * ROWS_W

    @pl.loop(0, ROWS_W // CH)
    def _chunk(ci):
        row = rbase + ci * CH
        pltpu.sync_copy(in_hbm.at[pl.ds((SC_BASE + row) * D, CHW)], xbuf)
        pltpu.sync_copy(emb_hbm.at[pl.ds(row * D, CHW)], ebuf)

        @pl.loop(0, CHW // 16, unroll=8)
        def _add(i):
            s = pl.ds(i * 16, 16)
            xbuf[s] = xbuf[s] + ebuf[s]

        pltpu.sync_copy(xbuf, out_hbm.at[pl.ds(row * D, CHW)])


def _tc_full(inputs, emb3, BL=256):
    return pl.pallas_call(
        _tc_body,
        grid=(L // BL,),
        in_specs=[
            pl.BlockSpec((B, BL, D), lambda j: (0, j, 0)),
            pl.BlockSpec((1, BL, D), lambda j: (0, j, 0)),
        ],
        out_specs=pl.BlockSpec((B, BL, D), lambda j: (0, j, 0)),
        out_shape=jax.ShapeDtypeStruct((B, L, D), inputs.dtype),
    )(inputs, emb3)


POS_W = L // NW
SCCH = 8
NCHUNK = POS_W // SCCH
NBUF = 3


@functools.partial(
    pl.kernel,
    out_type=jax.ShapeDtypeStruct((B, L, D), jnp.float32),
    mesh=plsc.VectorSubcoreMesh(core_axis_name="c", subcore_axis_name="s"),
    scratch_types=[
        pltpu.VMEM((NBUF, B, SCCH, D), jnp.float32),
        pltpu.VMEM((NBUF, SCCH, D), jnp.float32),
        pltpu.SemaphoreType.DMA((NBUF,)),
        pltpu.SemaphoreType.DMA((NBUF,)),
        pltpu.SemaphoreType.DMA((NBUF,)),
    ],
)
def _sc_full(in_hbm, emb_hbm, out_hbm, xbuf, ebuf, xsem, esem, osem):
    wid = lax.axis_index("s") * NC + lax.axis_index("c")
    wbase = wid * POS_W

    xd = [None] * NBUF
    ed = [None] * NBUF
    od = [None] * NBUF

    def start_in(g):
        slot = g % NBUF
        pos = wbase + g * SCCH
        ed[slot] = pltpu.async_copy(
            emb_hbm.at[pl.ds(pos, SCCH)], ebuf.at[slot], esem.at[slot])
        xd[slot] = pltpu.async_copy(
            in_hbm.at[:, pl.ds(pos, SCCH), :], xbuf.at[slot], xsem.at[slot])

    for g in range(min(NBUF - 1, NCHUNK)):
        start_in(g)

    for g in range(NCHUNK):
        slot = g % NBUF
        nxt = g + NBUF - 1
        if nxt < NCHUNK:
            nslot = nxt % NBUF
            if od[nslot] is not None:
                od[nslot].wait()
            start_in(nxt)
        ed[slot].wait()
        xd[slot].wait()

        if False:
            @pl.loop(0, SCCH)
            def _row(j):
                @pl.loop(0, D // 16, unroll=8)
                def _vec(k):
                    s = pl.ds(k * 16, 16)
                    e = ebuf[slot, j, s]
                    for b in range(B):
                        xbuf[slot, b, j, s] = xbuf[slot, b, j, s] + e

        pos = wbase + g * SCCH
        od[slot] = pltpu.async_copy(
            xbuf.at[slot], out_hbm.at[:, pl.ds(pos, SCCH), :], osem.at[slot])

    for g in range(max(0, NCHUNK - NBUF), NCHUNK):
        od[g % NBUF].wait()


def kernel(inputs, emb_table):
    return _sc_full(inputs, emb_table)


# speedup vs baseline: 1.8009x; 1.6892x over previous
import functools

import jax
import jax.numpy as jnp
from jax import lax
from jax.experimental import pallas as pl
from jax.experimental.pallas import tpu as pltpu
from jax.experimental.pallas import tpu_sc as plsc

B, L, D = 4, 2048, 1024
SC_BATCHES = 1
TC_B = B - SC_BATCHES
NC, NS = 2, 16
NW = NC * NS
SC_ROWS = SC_BATCHES * L
ROWS_W = SC_ROWS // NW
CH = 16
CHW = CH * D
SC_BASE = TC_B * L


def _tc_body(x_ref, e_ref, o_ref):
    o_ref[...] = x_ref[...] + e_ref[...]


def _tc_add(inputs, emb3):
    BL = 512
    return pl.pallas_call(
        _tc_body,
        grid=(L // BL,),
        in_specs=[
            pl.BlockSpec((TC_B, BL, D), lambda j: (0, j, 0)),
            pl.BlockSpec((1, BL, D), lambda j: (0, j, 0)),
        ],
        out_specs=pl.BlockSpec((TC_B, BL, D), lambda j: (0, j, 0)),
        out_shape=jax.ShapeDtypeStruct((TC_B, L, D), inputs.dtype),
    )(inputs, emb3)


@functools.partial(
    pl.kernel,
    out_type=jax.ShapeDtypeStruct((SC_ROWS * D,), jnp.float32),
    mesh=plsc.VectorSubcoreMesh(core_axis_name="c", subcore_axis_name="s"),
    scratch_types=[
        pltpu.VMEM((CHW,), jnp.float32),
        pltpu.VMEM((CHW,), jnp.float32),
    ],
)
def _sc_add(in_hbm, emb_hbm, out_hbm, xbuf, ebuf):
    wid = lax.axis_index("s") * NC + lax.axis_index("c")
    rbase = wid * ROWS_W

    @pl.loop(0, ROWS_W // CH)
    def _chunk(ci):
        row = rbase + ci * CH
        pltpu.sync_copy(in_hbm.at[pl.ds((SC_BASE + row) * D, CHW)], xbuf)
        pltpu.sync_copy(emb_hbm.at[pl.ds(row * D, CHW)], ebuf)

        @pl.loop(0, CHW // 16, unroll=8)
        def _add(i):
            s = pl.ds(i * 16, 16)
            xbuf[s] = xbuf[s] + ebuf[s]

        pltpu.sync_copy(xbuf, out_hbm.at[pl.ds(row * D, CHW)])


def _tc_full(inputs, emb3, BL=256):
    return pl.pallas_call(
        _tc_body,
        grid=(L // BL,),
        in_specs=[
            pl.BlockSpec((B, BL, D), lambda j: (0, j, 0)),
            pl.BlockSpec((1, BL, D), lambda j: (0, j, 0)),
        ],
        out_specs=pl.BlockSpec((B, BL, D), lambda j: (0, j, 0)),
        out_shape=jax.ShapeDtypeStruct((B, L, D), inputs.dtype),
    )(inputs, emb3)


POS_W = L // NW
SCCH = 8
NCHUNK = POS_W // SCCH
NBUF = 3


@functools.partial(
    pl.kernel,
    out_type=jax.ShapeDtypeStruct((B, L, D), jnp.float32),
    mesh=plsc.VectorSubcoreMesh(core_axis_name="c", subcore_axis_name="s"),
    scratch_types=[
        pltpu.VMEM((NBUF, B, SCCH, D), jnp.float32),
        pltpu.VMEM((NBUF, SCCH, D), jnp.float32),
        pltpu.SemaphoreType.DMA((NBUF,)),
        pltpu.SemaphoreType.DMA((NBUF,)),
        pltpu.SemaphoreType.DMA((NBUF,)),
    ],
)
def _sc_full(in_hbm, emb_hbm, out_hbm, xbuf, ebuf, xsem, esem, osem):
    wid = lax.axis_index("s") * NC + lax.axis_index("c")
    wbase = wid * POS_W

    xd = [None] * NBUF
    ed = [None] * NBUF
    od = [None] * NBUF

    def start_in(g):
        slot = g % NBUF
        pos = wbase + g * SCCH
        ed[slot] = pltpu.async_copy(
            emb_hbm.at[pl.ds(pos, SCCH)], ebuf.at[slot], esem.at[slot])
        xd[slot] = pltpu.async_copy(
            in_hbm.at[:, pl.ds(pos, SCCH), :], xbuf.at[slot], xsem.at[slot])

    for g in range(min(NBUF - 1, NCHUNK)):
        start_in(g)

    for g in range(NCHUNK):
        slot = g % NBUF
        nxt = g + NBUF - 1
        if nxt < NCHUNK:
            nslot = nxt % NBUF
            if od[nslot] is not None:
                od[nslot].wait()
            start_in(nxt)
        ed[slot].wait()
        xd[slot].wait()

        if False:
            @pl.loop(0, SCCH)
            def _row(j):
                @pl.loop(0, D // 16, unroll=8)
                def _vec(k):
                    s = pl.ds(k * 16, 16)
                    e = ebuf[slot, j, s]
                    for b in range(B):
                        xbuf[slot, b, j, s] = xbuf[slot, b, j, s] + e

        pos = wbase + g * SCCH
        od[slot] = pltpu.async_copy(
            xbuf.at[slot], out_hbm.at[:, pl.ds(pos, SCCH), :], osem.at[slot])

    for g in range(max(0, NCHUNK - NBUF), NCHUNK):
        od[g % NBUF].wait()


def kernel(inputs, emb_table):
    return _tc_full(inputs, emb_table[None], BL=128)


# speedup vs baseline: 1.9296x; 1.0714x over previous
import jax
import jax.numpy as jnp
from jax.experimental import pallas as pl

_BL = 256


def _add_body(x_ref, e_ref, o_ref):
    o_ref[...] = x_ref[...] + e_ref[...]


def kernel(inputs, emb_table):
    B, L, D = inputs.shape
    emb3 = emb_table[:L][None]
    return pl.pallas_call(
        _add_body,
        grid=(L // _BL,),
        in_specs=[
            pl.BlockSpec((B, _BL, D), lambda j: (0, j, 0)),
            pl.BlockSpec((1, _BL, D), lambda j: (0, j, 0)),
        ],
        out_specs=pl.BlockSpec((B, _BL, D), lambda j: (0, j, 0)),
        out_shape=jax.ShapeDtypeStruct((B, L, D), inputs.dtype),
    )(inputs, emb3)
